# baseline (speedup 1.0000x reference)
import functools

import jax
import jax.numpy as jnp
from jax import lax
from jax.experimental import pallas as pl
from jax.experimental.pallas import tpu as pltpu
from jax.experimental.pallas import tpu_sc as plsc

NS = 26
NFA = 13
NFB = NS - NFA
ND = 13
B = 16384
V = 100000
VPAD = 100352
NC = 2
NSUB = 16
NW = NC * NSUB
BPW = B // NW
CHUNKS = BPW // 16

_mesh = plsc.VectorSubcoreMesh(core_axis_name="c", subcore_axis_name="s")


def _phase_body(nf, with_dense, args):
    it = iter(args)
    if with_dense:
        wd_hbm = next(it)
    idx_hbms = [next(it) for _ in range(nf)]
    if with_dense:
        extra_hbms = [next(it) for _ in range(ND)]
    else:
        extra_hbms = [next(it)]
    table_hbm = next(it)
    out_hbm = next(it)
    rest = list(it)

    nextra = ND if with_dense else 1
    gidx_v = rest[0]
    gath_v = rest[1]
    extra_vs = rest[2:2 + nextra]
    wd_v, acc_v, sem_in, sem_g = rest[2 + nextra:]

    wid = lax.axis_index("s") * NC + lax.axis_index("c")
    base = wid * BPW

    copies = []
    if with_dense:
        copies.append(pltpu.async_copy(wd_hbm, wd_v, sem_in))
    for i in range(nf):
        copies.append(
            pltpu.async_copy(
                idx_hbms[i].at[pl.ds(base, BPW)],
                gidx_v.at[pl.ds(i * BPW, BPW)],
                sem_in,
            )
        )
    for d in range(nextra):
        copies.append(
            pltpu.async_copy(extra_hbms[d].at[pl.ds(base, BPW)], extra_vs[d], sem_in)
        )
    for c in copies:
        c.wait()

    def off_body(c, _):
        o = c * 16
        f = c // CHUNKS
        gidx_v[pl.ds(o, 16)] = gidx_v[pl.ds(o, 16)] + f * VPAD
        return _

    lax.fori_loop(CHUNKS, nf * CHUNKS, off_body, None)

    pltpu.async_copy(table_hbm.at[gidx_v], gath_v, sem_g).wait()

    def chunk_body(c, _):
        o = c * 16
        acc = gath_v[pl.ds(o, 16)]
        for i in range(1, nf):
            acc = acc + gath_v[pl.ds(i * BPW + o, 16)]
        if with_dense:
            for d in range(ND):
                acc = acc + extra_vs[d][pl.ds(o, 16)] * wd_v[d, :]
        else:
            acc = acc + extra_vs[0][pl.ds(o, 16)]
        acc_v[pl.ds(o, 16)] = acc
        return _

    lax.fori_loop(0, CHUNKS, chunk_body, None)

    pltpu.sync_copy(acc_v, out_hbm.at[pl.ds(base, BPW)])


def _make_phase(nf, with_dense):
    nextra = ND if with_dense else 1
    scratch = (
        [pltpu.VMEM((nf * BPW,), jnp.int32)]
        + [pltpu.VMEM((nf * BPW,), jnp.float32)]
        + [pltpu.VMEM((BPW,), jnp.float32) for _ in range(nextra)]
        + [
            pltpu.VMEM((ND, 16), jnp.float32),
            pltpu.VMEM((BPW,), jnp.float32),
            pltpu.SemaphoreType.DMA,
            pltpu.SemaphoreType.DMA,
        ]
    )

    @functools.partial(
        pl.kernel,
        mesh=_mesh,
        out_type=jax.ShapeDtypeStruct((B,), jnp.float32),
        scratch_types=scratch,
    )
    def phase(*args):
        _phase_body(nf, with_dense, args)

    return phase


_phase_a = _make_phase(NFA, True)
_phase_b = _make_phase(NFB, False)


def kernel(sparse_0, sparse_1, sparse_2, sparse_3, sparse_4, sparse_5, sparse_6, sparse_7, sparse_8, sparse_9, sparse_10, sparse_11, sparse_12, sparse_13, sparse_14, sparse_15, sparse_16, sparse_17, sparse_18, sparse_19, sparse_20, sparse_21, sparse_22, sparse_23, sparse_24, sparse_25, dense_0, dense_1, dense_2, dense_3, dense_4, dense_5, dense_6, dense_7, dense_8, dense_9, dense_10, dense_11, dense_12, W_0, W_1, W_2, W_3, W_4, W_5, W_6, W_7, W_8, W_9, W_10, W_11, W_12, W_13, W_14, W_15, W_16, W_17, W_18, W_19, W_20, W_21, W_22, W_23, W_24, W_25, Wd):
    sparse = [sparse_0, sparse_1, sparse_2, sparse_3, sparse_4, sparse_5,
              sparse_6, sparse_7, sparse_8, sparse_9, sparse_10, sparse_11,
              sparse_12, sparse_13, sparse_14, sparse_15, sparse_16,
              sparse_17, sparse_18, sparse_19, sparse_20, sparse_21,
              sparse_22, sparse_23, sparse_24, sparse_25]
    dense = [dense_0, dense_1, dense_2, dense_3, dense_4, dense_5, dense_6,
             dense_7, dense_8, dense_9, dense_10, dense_11, dense_12]
    tables = [W_0, W_1, W_2, W_3, W_4, W_5, W_6, W_7, W_8, W_9, W_10, W_11,
              W_12, W_13, W_14, W_15, W_16, W_17, W_18, W_19, W_20, W_21,
              W_22, W_23, W_24, W_25]

    idx1 = [s.reshape(B) for s in sparse]
    dense1 = [d.reshape(B) for d in dense]
    wd16 = jnp.broadcast_to(Wd, (ND, 16))
    flat_tables = [
        jnp.pad(w, ((0, VPAD - V), (0, 0))).reshape(VPAD) for w in tables
    ]
    table_a = jnp.concatenate(flat_tables[:NFA])
    table_b = jnp.concatenate(flat_tables[NFA:])

    part = _phase_a(wd16, *idx1[:NFA], *dense1, table_a)
    out = _phase_b(*idx1[NFA:], part, table_b)
    return out.reshape(B, 1)


# speedup vs baseline: 1.0511x; 1.0511x over previous
import functools

import jax
import jax.numpy as jnp
from jax import lax
from jax.experimental import pallas as pl
from jax.experimental.pallas import tpu as pltpu
from jax.experimental.pallas import tpu_sc as plsc

NS = 26
NFA = 13
NFB = NS - NFA
ND = 13
B = 16384
V = 100000
VPAD = 100352
NC = 2
NSUB = 16
NW = NC * NSUB
BPW = B // NW
CHUNKS = BPW // 16

_mesh = plsc.VectorSubcoreMesh(core_axis_name="c", subcore_axis_name="s")


def _phase_body(nf, with_dense, args):
    it = iter(args)
    if with_dense:
        wd_hbm = next(it)
    idx_hbms = [next(it) for _ in range(nf)]
    if with_dense:
        extra_hbms = [next(it) for _ in range(ND)]
    else:
        extra_hbms = [next(it)]
    table_hbm = next(it)
    out_hbm = next(it)
    rest = list(it)

    nextra = ND if with_dense else 1
    idx_vs = rest[:nf]
    gath_vs = rest[nf:2 * nf]
    extra_vs = rest[2 * nf:2 * nf + nextra]
    wd_v, acc_v, sem_in, sem_g = rest[2 * nf + nextra:]

    wid = lax.axis_index("s") * NC + lax.axis_index("c")
    base = wid * BPW

    copies = []
    if with_dense:
        copies.append(pltpu.async_copy(wd_hbm, wd_v, sem_in))
    for i in range(nf):
        copies.append(
            pltpu.async_copy(idx_hbms[i].at[pl.ds(base, BPW)], idx_vs[i], sem_in)
        )
    for d in range(nextra):
        copies.append(
            pltpu.async_copy(extra_hbms[d].at[pl.ds(base, BPW)], extra_vs[d], sem_in)
        )
    for c in copies:
        c.wait()

    gathers = []
    for i in range(nf):
        gathers.append(
            pltpu.async_copy(
                table_hbm.at[pl.ds(i * VPAD, VPAD)].at[idx_vs[i]],
                gath_vs[i],
                sem_g,
            )
        )
    for c in gathers:
        c.wait()

    def chunk_body(c, _):
        o = c * 16
        acc = gath_vs[0][pl.ds(o, 16)]
        for i in range(1, nf):
            acc = acc + gath_vs[i][pl.ds(o, 16)]
        if with_dense:
            for d in range(ND):
                acc = acc + extra_vs[d][pl.ds(o, 16)] * wd_v[d, :]
        else:
            acc = acc + extra_vs[0][pl.ds(o, 16)]
        acc_v[pl.ds(o, 16)] = acc
        return _

    lax.fori_loop(0, CHUNKS, chunk_body, None)

    pltpu.sync_copy(acc_v, out_hbm.at[pl.ds(base, BPW)])


def _make_phase(nf, with_dense):
    nextra = ND if with_dense else 1
    scratch = (
        [pltpu.VMEM((BPW,), jnp.int32) for _ in range(nf)]
        + [pltpu.VMEM((BPW,), jnp.float32) for _ in range(nf)]
        + [pltpu.VMEM((BPW,), jnp.float32) for _ in range(nextra)]
        + [
            pltpu.VMEM((ND, 16), jnp.float32),
            pltpu.VMEM((BPW,), jnp.float32),
            pltpu.SemaphoreType.DMA,
            pltpu.SemaphoreType.DMA,
        ]
    )

    @functools.partial(
        pl.kernel,
        mesh=_mesh,
        out_type=jax.ShapeDtypeStruct((B,), jnp.float32),
        scratch_types=scratch,
    )
    def phase(*args):
        _phase_body(nf, with_dense, args)

    return phase


_phase_a = _make_phase(NFA, True)
_phase_b = _make_phase(NFB, False)


def kernel(sparse_0, sparse_1, sparse_2, sparse_3, sparse_4, sparse_5, sparse_6, sparse_7, sparse_8, sparse_9, sparse_10, sparse_11, sparse_12, sparse_13, sparse_14, sparse_15, sparse_16, sparse_17, sparse_18, sparse_19, sparse_20, sparse_21, sparse_22, sparse_23, sparse_24, sparse_25, dense_0, dense_1, dense_2, dense_3, dense_4, dense_5, dense_6, dense_7, dense_8, dense_9, dense_10, dense_11, dense_12, W_0, W_1, W_2, W_3, W_4, W_5, W_6, W_7, W_8, W_9, W_10, W_11, W_12, W_13, W_14, W_15, W_16, W_17, W_18, W_19, W_20, W_21, W_22, W_23, W_24, W_25, Wd):
    sparse = [sparse_0, sparse_1, sparse_2, sparse_3, sparse_4, sparse_5,
              sparse_6, sparse_7, sparse_8, sparse_9, sparse_10, sparse_11,
              sparse_12, sparse_13, sparse_14, sparse_15, sparse_16,
              sparse_17, sparse_18, sparse_19, sparse_20, sparse_21,
              sparse_22, sparse_23, sparse_24, sparse_25]
    dense = [dense_0, dense_1, dense_2, dense_3, dense_4, dense_5, dense_6,
             dense_7, dense_8, dense_9, dense_10, dense_11, dense_12]
    tables = [W_0, W_1, W_2, W_3, W_4, W_5, W_6, W_7, W_8, W_9, W_10, W_11,
              W_12, W_13, W_14, W_15, W_16, W_17, W_18, W_19, W_20, W_21,
              W_22, W_23, W_24, W_25]

    idx1 = [s.reshape(B) for s in sparse]
    dense1 = [d.reshape(B) for d in dense]
    wd16 = jnp.broadcast_to(Wd, (ND, 16))
    flat_tables = [
        jnp.pad(w, ((0, VPAD - V), (0, 0))).reshape(VPAD) for w in tables
    ]
    table_a = jnp.concatenate(flat_tables[:NFA])
    table_b = jnp.concatenate(flat_tables[NFA:])

    part = _phase_a(wd16, *idx1[:NFA], *dense1, table_a)
    out = _phase_b(*idx1[NFA:], part, table_b)
    return out.reshape(B, 1)
